# Initial kernel scaffold; baseline (speedup 1.0000x reference)
import jax
import jax.numpy as jnp
from jax.experimental import pallas as pl


def kernel(x, edge_index, W1, b1, W2, b2):
    raise NotImplementedError("write your pallas kernel here")


# speedup vs baseline: 20.6761x; 20.6761x over previous
import functools

import jax
import jax.numpy as jnp
from jax import lax
from jax.experimental import pallas as pl
from jax.experimental.pallas import tpu as pltpu
from jax.experimental.pallas import tpu_sc as plsc

N = 10000
E = 320000
D_IN = 128
D_HID = 16

NC = 2
NS = 16
NW = NC * NS
BLK = 128
EPW = 10240
E_PAD = EPW * NW
NB = EPW // BLK
N_ACC = 10240
RPT = N_ACC // NS
ROWS_TC = 1000


def _edge_pass(width: int, do_gather: bool):
  def body(*refs):
    if do_gather:
      (table_hbm, src_hbm, dst_hbm, zeros_hbm, out_hbm,
       sbuf, dbuf, rows, zbuf, acc, sem) = refs
    else:
      (dst_hbm, ones_hbm, zeros_hbm, out_hbm,
       dbuf, rows, zbuf, acc, sem) = refs
    c = lax.axis_index("c")
    s = lax.axis_index("s")
    wid = s * NC + c

    pltpu.sync_copy(zeros_hbm, zbuf)
    pltpu.sync_copy(zbuf, acc.at[pl.ds(s * RPT, RPT)])
    if not do_gather:
      pltpu.sync_copy(ones_hbm, rows)
    plsc.subcore_barrier()

    @pl.loop(0, NB)
    def _(j):
      base = wid * EPW + j * BLK
      if do_gather:
        pltpu.sync_copy(src_hbm.at[pl.ds(base, BLK)], sbuf)
        pltpu.async_copy(table_hbm.at[sbuf], rows, sem).wait()
      pltpu.sync_copy(dst_hbm.at[pl.ds(base, BLK)], dbuf)
      pltpu.sync_copy(rows, acc.at[dbuf], add=True)

    plsc.subcore_barrier()
    pltpu.sync_copy(acc.at[pl.ds(s * RPT, RPT)],
                    out_hbm.at[c, pl.ds(s * RPT, RPT)])

  scratch = []
  if do_gather:
    scratch.append(pltpu.VMEM((BLK,), jnp.int32))
  scratch += [
      pltpu.VMEM((BLK,), jnp.int32),
      pltpu.VMEM((BLK, width), jnp.float32),
      pltpu.VMEM((RPT, width), jnp.float32),
      pltpu.VMEM_SHARED((N_ACC, width), jnp.float32),
      pltpu.SemaphoreType.DMA,
  ]
  return pl.kernel(
      body,
      out_type=jax.ShapeDtypeStruct((NC, N_ACC, width), jnp.float32),
      mesh=plsc.VectorSubcoreMesh(core_axis_name="c", subcore_axis_name="s"),
      scratch_types=scratch,
      compiler_params=pltpu.CompilerParams(use_tc_tiling_on_sc=False),
  )


def _mm_body(x_ref, w_ref, o_ref):
  o_ref[...] = jnp.dot(x_ref[...], w_ref[...],
                       preferred_element_type=jnp.float32)


_matmul = pl.pallas_call(
    _mm_body,
    grid=(N // ROWS_TC,),
    in_specs=[
        pl.BlockSpec((ROWS_TC, D_IN), lambda i: (i, 0)),
        pl.BlockSpec((D_IN, D_HID), lambda i: (0, 0)),
    ],
    out_specs=pl.BlockSpec((ROWS_TC, D_HID), lambda i: (i, 0)),
    out_shape=jax.ShapeDtypeStruct((N, D_HID), jnp.float32),
)


def _norm_body(d0_ref, d1_ref, h_ref, dis_ref, u1_ref):
  deg = d0_ref[...] + d1_ref[...] + 1.0
  dis = lax.rsqrt(deg)
  dis_ref[...] = dis
  u1_ref[...] = h_ref[...] * dis


_norm = pl.pallas_call(
    _norm_body,
    grid=(N // ROWS_TC,),
    in_specs=[
        pl.BlockSpec((ROWS_TC, 1), lambda i: (i, 0)),
        pl.BlockSpec((ROWS_TC, 1), lambda i: (i, 0)),
        pl.BlockSpec((ROWS_TC, D_HID), lambda i: (i, 0)),
    ],
    out_specs=[
        pl.BlockSpec((ROWS_TC, 1), lambda i: (i, 0)),
        pl.BlockSpec((ROWS_TC, D_HID), lambda i: (i, 0)),
    ],
    out_shape=[
        jax.ShapeDtypeStruct((N, 1), jnp.float32),
        jax.ShapeDtypeStruct((N, D_HID), jnp.float32),
    ],
)


def _layer1_out_body(s0_ref, s1_ref, u1_ref, dis_ref, b1_ref, w2_ref, u2_ref):
  s1 = s0_ref[...] + s1_ref[...] + u1_ref[...]
  out1 = dis_ref[...] * s1 + b1_ref[...]
  h1 = jnp.maximum(out1, 0.0)
  z = jnp.sum(h1 * w2_ref[...], axis=1, keepdims=True)
  u2_ref[...] = dis_ref[...] * z


_layer1_out = pl.pallas_call(
    _layer1_out_body,
    grid=(N // ROWS_TC,),
    in_specs=[
        pl.BlockSpec((ROWS_TC, D_HID), lambda i: (i, 0)),
        pl.BlockSpec((ROWS_TC, D_HID), lambda i: (i, 0)),
        pl.BlockSpec((ROWS_TC, D_HID), lambda i: (i, 0)),
        pl.BlockSpec((ROWS_TC, 1), lambda i: (i, 0)),
        pl.BlockSpec((1, D_HID), lambda i: (0, 0)),
        pl.BlockSpec((1, D_HID), lambda i: (0, 0)),
    ],
    out_specs=pl.BlockSpec((ROWS_TC, 1), lambda i: (i, 0)),
    out_shape=jax.ShapeDtypeStruct((N, 1), jnp.float32),
)


def _layer2_out_body(s0_ref, s1_ref, u2_ref, dis_ref, b2_ref, o_ref):
  s2 = s0_ref[...] + s1_ref[...] + u2_ref[...]
  o_ref[...] = dis_ref[...] * s2 + b2_ref[...]


_layer2_out = pl.pallas_call(
    _layer2_out_body,
    grid=(N // ROWS_TC,),
    in_specs=[
        pl.BlockSpec((ROWS_TC, 1), lambda i: (i, 0)),
        pl.BlockSpec((ROWS_TC, 1), lambda i: (i, 0)),
        pl.BlockSpec((ROWS_TC, 1), lambda i: (i, 0)),
        pl.BlockSpec((ROWS_TC, 1), lambda i: (i, 0)),
        pl.BlockSpec((1, 1), lambda i: (0, 0)),
    ],
    out_specs=pl.BlockSpec((ROWS_TC, 1), lambda i: (i, 0)),
    out_shape=jax.ShapeDtypeStruct((N, 1), jnp.float32),
)

_deg_pass = _edge_pass(1, do_gather=False)
_agg16_pass = _edge_pass(D_HID, do_gather=True)
_agg1_pass = _edge_pass(1, do_gather=True)


@jax.jit
def kernel(x, edge_index, W1, b1, W2, b2):
  src = edge_index[0]
  dst = edge_index[1]
  pad = jnp.full((E_PAD - E,), N, dtype=jnp.int32)
  srcp = jnp.concatenate([src, pad])
  dstp = jnp.concatenate([dst, pad])

  ones1 = jnp.ones((BLK, 1), jnp.float32)
  zeros1 = jnp.zeros((RPT, 1), jnp.float32)
  zeros16 = jnp.zeros((RPT, D_HID), jnp.float32)

  degp = _deg_pass(dstp, ones1, zeros1)
  h = _matmul(x, W1)

  dis, u1 = _norm(degp[0, :N], degp[1, :N], h)

  u1_pad = jnp.zeros((N_ACC, D_HID), jnp.float32).at[:N].set(u1)
  s1p = _agg16_pass(u1_pad, srcp, dstp, zeros16)

  u2 = _layer1_out(s1p[0, :N], s1p[1, :N], u1, dis,
                   b1.reshape(1, D_HID), W2.reshape(1, D_HID))

  u2_pad = jnp.zeros((N_ACC, 1), jnp.float32).at[:N].set(u2)
  s2p = _agg1_pass(u2_pad, srcp, dstp, zeros1)

  return _layer2_out(s2p[0, :N], s2p[1, :N], u2, dis, b2.reshape(1, 1))


# speedup vs baseline: 26.7486x; 1.2937x over previous
import functools

import jax
import jax.numpy as jnp
from jax import lax
from jax.experimental import pallas as pl
from jax.experimental.pallas import tpu as pltpu
from jax.experimental.pallas import tpu_sc as plsc

N = 10000
E = 320000
D_IN = 128
D_HID = 16

NC = 2
NS = 16
NW = NC * NS
BLK = 128
EPW = 10240
E_PAD = EPW * NW
NB = EPW // BLK
N_ACC = 10240
RPT = N_ACC // NS
ROWS_TC = 1000


def _edge_pass(width: int, do_gather: bool):
  K = 8

  def body(*refs):
    if do_gather:
      (table_hbm, src_hbm, dst_hbm, zeros_hbm, out_hbm, *rest) = refs
      sbufs = rest[0:K]
      dbufs = rest[K:2 * K]
      rows = rest[2 * K:3 * K]
      zbuf, acc, gsem, ssem = rest[3 * K:]
    else:
      (dst_hbm, ones_hbm, zeros_hbm, out_hbm, *rest) = refs
      dbufs = rest[0:K]
      ones_v = rest[K]
      zbuf, acc, ssem = rest[K + 1:]
    c = lax.axis_index("c")
    s = lax.axis_index("s")
    wid = s * NC + c

    pltpu.sync_copy(zeros_hbm, zbuf)
    pltpu.sync_copy(zbuf, acc.at[pl.ds(s * RPT, RPT)])
    if not do_gather:
      pltpu.sync_copy(ones_hbm, ones_v)
    plsc.subcore_barrier()

    @pl.loop(0, NB // K)
    def _(g):
      base = wid * EPW + g * (K * BLK)
      if do_gather:
        for b in range(K):
          pltpu.sync_copy(src_hbm.at[pl.ds(base + b * BLK, BLK)], sbufs[b])
        for b in range(K):
          pltpu.async_copy(table_hbm.at[sbufs[b]], rows[b], gsem)
        for b in range(K):
          pltpu.sync_copy(dst_hbm.at[pl.ds(base + b * BLK, BLK)], dbufs[b])
        for b in range(K):
          pltpu.make_async_copy(table_hbm.at[sbufs[b]], rows[b], gsem).wait()
        for b in range(K):
          pltpu.async_copy(rows[b], acc.at[dbufs[b]], ssem, add=True)
        for b in range(K):
          pltpu.make_async_copy(rows[b], acc.at[dbufs[b]], ssem).wait()
      else:
        for b in range(K):
          pltpu.sync_copy(dst_hbm.at[pl.ds(base + b * BLK, BLK)], dbufs[b])
        for b in range(K):
          pltpu.async_copy(ones_v, acc.at[dbufs[b]], ssem, add=True)
        for b in range(K):
          pltpu.make_async_copy(ones_v, acc.at[dbufs[b]], ssem).wait()

    plsc.subcore_barrier()
    pltpu.sync_copy(acc.at[pl.ds(s * RPT, RPT)],
                    out_hbm.at[c, pl.ds(s * RPT, RPT)])

  scratch = []
  if do_gather:
    scratch += [pltpu.VMEM((BLK,), jnp.int32) for _ in range(K)]
  scratch += [pltpu.VMEM((BLK,), jnp.int32) for _ in range(K)]
  if do_gather:
    scratch += [pltpu.VMEM((BLK, width), jnp.float32) for _ in range(K)]
  else:
    scratch += [pltpu.VMEM((BLK, width), jnp.float32)]
  scratch += [
      pltpu.VMEM((RPT, width), jnp.float32),
      pltpu.VMEM_SHARED((N_ACC, width), jnp.float32),
      pltpu.SemaphoreType.DMA,
  ]
  if do_gather:
    scratch.append(pltpu.SemaphoreType.DMA)
  return pl.kernel(
      body,
      out_type=jax.ShapeDtypeStruct((NC, N_ACC, width), jnp.float32),
      mesh=plsc.VectorSubcoreMesh(core_axis_name="c", subcore_axis_name="s"),
      scratch_types=scratch,
      compiler_params=pltpu.CompilerParams(use_tc_tiling_on_sc=False),
  )


def _mm_norm_body(x_ref, w_ref, d0_ref, d1_ref, dis_ref, u1_ref):
  h = jnp.dot(x_ref[...], w_ref[...], preferred_element_type=jnp.float32)
  deg = d0_ref[...] + d1_ref[...] + 1.0
  dis = lax.rsqrt(deg)
  dis_ref[...] = dis
  u1_ref[...] = h * dis


_mm_norm = pl.pallas_call(
    _mm_norm_body,
    grid=(N // ROWS_TC,),
    in_specs=[
        pl.BlockSpec((ROWS_TC, D_IN), lambda i: (i, 0)),
        pl.BlockSpec((D_IN, D_HID), lambda i: (0, 0)),
        pl.BlockSpec((ROWS_TC, 1), lambda i: (i, 0)),
        pl.BlockSpec((ROWS_TC, 1), lambda i: (i, 0)),
    ],
    out_specs=[
        pl.BlockSpec((ROWS_TC, 1), lambda i: (i, 0)),
        pl.BlockSpec((ROWS_TC, D_HID), lambda i: (i, 0)),
    ],
    out_shape=[
        jax.ShapeDtypeStruct((N, 1), jnp.float32),
        jax.ShapeDtypeStruct((N, D_HID), jnp.float32),
    ],
)


def _layer1_out_body(s0_ref, s1_ref, u1_ref, dis_ref, b1_ref, w2_ref, u2_ref):
  s1 = s0_ref[...] + s1_ref[...] + u1_ref[...]
  out1 = dis_ref[...] * s1 + b1_ref[...]
  h1 = jnp.maximum(out1, 0.0)
  z = jnp.sum(h1 * w2_ref[...], axis=1, keepdims=True)
  u2_ref[...] = dis_ref[...] * z


_layer1_out = pl.pallas_call(
    _layer1_out_body,
    grid=(N // ROWS_TC,),
    in_specs=[
        pl.BlockSpec((ROWS_TC, D_HID), lambda i: (i, 0)),
        pl.BlockSpec((ROWS_TC, D_HID), lambda i: (i, 0)),
        pl.BlockSpec((ROWS_TC, D_HID), lambda i: (i, 0)),
        pl.BlockSpec((ROWS_TC, 1), lambda i: (i, 0)),
        pl.BlockSpec((1, D_HID), lambda i: (0, 0)),
        pl.BlockSpec((1, D_HID), lambda i: (0, 0)),
    ],
    out_specs=pl.BlockSpec((ROWS_TC, 1), lambda i: (i, 0)),
    out_shape=jax.ShapeDtypeStruct((N, 1), jnp.float32),
)


def _layer2_out_body(s0_ref, s1_ref, u2_ref, dis_ref, b2_ref, o_ref):
  s2 = s0_ref[...] + s1_ref[...] + u2_ref[...]
  o_ref[...] = dis_ref[...] * s2 + b2_ref[...]


_layer2_out = pl.pallas_call(
    _layer2_out_body,
    grid=(N // ROWS_TC,),
    in_specs=[
        pl.BlockSpec((ROWS_TC, 1), lambda i: (i, 0)),
        pl.BlockSpec((ROWS_TC, 1), lambda i: (i, 0)),
        pl.BlockSpec((ROWS_TC, 1), lambda i: (i, 0)),
        pl.BlockSpec((ROWS_TC, 1), lambda i: (i, 0)),
        pl.BlockSpec((1, 1), lambda i: (0, 0)),
    ],
    out_specs=pl.BlockSpec((ROWS_TC, 1), lambda i: (i, 0)),
    out_shape=jax.ShapeDtypeStruct((N, 1), jnp.float32),
)

_deg_pass = _edge_pass(1, do_gather=False)
_agg16_pass = _edge_pass(D_HID, do_gather=True)
_agg1_pass = _edge_pass(1, do_gather=True)


@jax.jit
def kernel(x, edge_index, W1, b1, W2, b2):
  src = edge_index[0]
  dst = edge_index[1]
  pad = jnp.full((E_PAD - E,), N, dtype=jnp.int32)
  srcp = jnp.concatenate([src, pad])
  dstp = jnp.concatenate([dst, pad])

  ones1 = jnp.ones((BLK, 1), jnp.float32)
  zeros1 = jnp.zeros((RPT, 1), jnp.float32)
  zeros16 = jnp.zeros((RPT, D_HID), jnp.float32)

  degp = _deg_pass(dstp, ones1, zeros1)
  dis, u1 = _mm_norm(x, W1, degp[0, :N], degp[1, :N])

  u1_pad = jnp.zeros((N_ACC, D_HID), jnp.float32).at[:N].set(u1)
  s1p = _agg16_pass(u1_pad, srcp, dstp, zeros16)

  u2 = _layer1_out(s1p[0, :N], s1p[1, :N], u1, dis,
                   b1.reshape(1, D_HID), W2.reshape(1, D_HID))

  u2_pad = jnp.zeros((N_ACC, 1), jnp.float32).at[:N].set(u2)
  s2p = _agg1_pass(u2_pad, srcp, dstp, zeros1)

  return _layer2_out(s2p[0, :N], s2p[1, :N], u2, dis, b2.reshape(1, 1))


# speedup vs baseline: 27.8454x; 1.0410x over previous
import functools

import jax
import jax.numpy as jnp
from jax import lax
from jax.experimental import pallas as pl
from jax.experimental.pallas import tpu as pltpu
from jax.experimental.pallas import tpu_sc as plsc

N = 10000
E = 320000
D_IN = 128
D_HID = 16

NC = 2
NS = 16
NW = NC * NS
BLK = 128
EPW = 10240
E_PAD = EPW * NW
NB = EPW // BLK
N_ACC = 10240
RPT = N_ACC // NS
ROWS_TC = 1000


def _edge_pass(width: int, do_gather: bool):
  K = 8

  def body(*refs):
    if do_gather:
      (table_hbm, src_hbm, dst_hbm, zeros_hbm, out_hbm, *rest) = refs
      banks = [(rest[0:K], rest[2 * K:3 * K], rest[4 * K:5 * K]),
               (rest[K:2 * K], rest[3 * K:4 * K], rest[5 * K:6 * K])]
      zbuf, acc, gsem, ssemA, ssemB = rest[6 * K:]
    else:
      (dst_hbm, ones_hbm, zeros_hbm, out_hbm, *rest) = refs
      banks = [(None, rest[0:K], None), (None, rest[K:2 * K], None)]
      ones_v = rest[2 * K]
      zbuf, acc, ssemA, ssemB = rest[2 * K + 1:]
    ssems = [ssemA, ssemB]
    c = lax.axis_index("c")
    s = lax.axis_index("s")
    wid = s * NC + c

    pltpu.sync_copy(zeros_hbm, zbuf)
    pltpu.sync_copy(zbuf, acc.at[pl.ds(s * RPT, RPT)])
    if not do_gather:
      pltpu.sync_copy(ones_hbm, ones_v)
    plsc.subcore_barrier()

    def prep(bank, g):
      sb, db, rw = bank
      base = wid * EPW + g * (K * BLK)
      if do_gather:
        for b in range(K):
          pltpu.sync_copy(src_hbm.at[pl.ds(base + b * BLK, BLK)], sb[b])
        for b in range(K):
          pltpu.async_copy(table_hbm.at[sb[b]], rw[b], gsem)
      for b in range(K):
        pltpu.sync_copy(dst_hbm.at[pl.ds(base + b * BLK, BLK)], db[b])
      if do_gather:
        for b in range(K):
          pltpu.make_async_copy(table_hbm.at[sb[b]], rw[b], gsem).wait()

    def fire(i):
      _, db, rw = banks[i]
      for b in range(K):
        pltpu.async_copy(rw[b] if do_gather else ones_v,
                         acc.at[db[b]], ssems[i], add=True)

    def drain(i):
      _, db, rw = banks[i]
      for b in range(K):
        pltpu.make_async_copy(rw[b] if do_gather else ones_v,
                              acc.at[db[b]], ssems[i]).wait()

    @pl.loop(0, NB // (2 * K))
    def _(i):
      prep(banks[0], 2 * i)

      @pl.when(i > 0)
      def _():
        drain(1)

      fire(0)
      prep(banks[1], 2 * i + 1)
      drain(0)
      fire(1)

    drain(1)

    plsc.subcore_barrier()
    pltpu.sync_copy(acc.at[pl.ds(s * RPT, RPT)],
                    out_hbm.at[c, pl.ds(s * RPT, RPT)])

  scratch = []
  if do_gather:
    scratch += [pltpu.VMEM((BLK,), jnp.int32) for _ in range(2 * K)]
  scratch += [pltpu.VMEM((BLK,), jnp.int32) for _ in range(2 * K)]
  if do_gather:
    scratch += [pltpu.VMEM((BLK, width), jnp.float32) for _ in range(2 * K)]
  else:
    scratch += [pltpu.VMEM((BLK, width), jnp.float32)]
  scratch += [
      pltpu.VMEM((RPT, width), jnp.float32),
      pltpu.VMEM_SHARED((N_ACC, width), jnp.float32),
  ]
  if do_gather:
    scratch.append(pltpu.SemaphoreType.DMA)
  scratch += [pltpu.SemaphoreType.DMA, pltpu.SemaphoreType.DMA]
  return pl.kernel(
      body,
      out_type=jax.ShapeDtypeStruct((NC, N_ACC, width), jnp.float32),
      mesh=plsc.VectorSubcoreMesh(core_axis_name="c", subcore_axis_name="s"),
      scratch_types=scratch,
      compiler_params=pltpu.CompilerParams(use_tc_tiling_on_sc=False),
  )


def _mm_body(x_ref, w_ref, h_ref):
  h_ref[...] = jnp.dot(x_ref[...], w_ref[...],
                       preferred_element_type=jnp.float32)


_matmul = pl.pallas_call(
    _mm_body,
    grid=(N // ROWS_TC,),
    in_specs=[
        pl.BlockSpec((ROWS_TC, D_IN), lambda i: (i, 0)),
        pl.BlockSpec((D_IN, D_HID), lambda i: (0, 0)),
    ],
    out_specs=pl.BlockSpec((ROWS_TC, D_HID), lambda i: (i, 0)),
    out_shape=jax.ShapeDtypeStruct((N, D_HID), jnp.float32),
)


def _norm_body(d0_ref, d1_ref, h_ref, dis_ref, u1_ref):
  deg = d0_ref[...] + d1_ref[...] + 1.0
  dis = lax.rsqrt(deg)
  dis_ref[...] = dis
  u1_ref[...] = h_ref[...] * dis


_norm = pl.pallas_call(
    _norm_body,
    grid=(N // ROWS_TC,),
    in_specs=[
        pl.BlockSpec((ROWS_TC, 1), lambda i: (i, 0)),
        pl.BlockSpec((ROWS_TC, 1), lambda i: (i, 0)),
        pl.BlockSpec((ROWS_TC, D_HID), lambda i: (i, 0)),
    ],
    out_specs=[
        pl.BlockSpec((ROWS_TC, 1), lambda i: (i, 0)),
        pl.BlockSpec((ROWS_TC, D_HID), lambda i: (i, 0)),
    ],
    out_shape=[
        jax.ShapeDtypeStruct((N, 1), jnp.float32),
        jax.ShapeDtypeStruct((N_ACC, D_HID), jnp.float32),
    ],
)


def _layer1_out_body(s0_ref, s1_ref, u1_ref, dis_ref, b1_ref, w2_ref, u2_ref):
  s1 = s0_ref[...] + s1_ref[...] + u1_ref[...]
  out1 = dis_ref[...] * s1 + b1_ref[...]
  h1 = jnp.maximum(out1, 0.0)
  z = jnp.sum(h1 * w2_ref[...], axis=1, keepdims=True)
  u2_ref[...] = dis_ref[...] * z


_layer1_out = pl.pallas_call(
    _layer1_out_body,
    grid=(N // ROWS_TC,),
    in_specs=[
        pl.BlockSpec((ROWS_TC, D_HID), lambda i: (i, 0)),
        pl.BlockSpec((ROWS_TC, D_HID), lambda i: (i, 0)),
        pl.BlockSpec((ROWS_TC, D_HID), lambda i: (i, 0)),
        pl.BlockSpec((ROWS_TC, 1), lambda i: (i, 0)),
        pl.BlockSpec((1, D_HID), lambda i: (0, 0)),
        pl.BlockSpec((1, D_HID), lambda i: (0, 0)),
    ],
    out_specs=pl.BlockSpec((ROWS_TC, 1), lambda i: (i, 0)),
    out_shape=jax.ShapeDtypeStruct((N_ACC, 1), jnp.float32),
)


def _layer2_out_body(s0_ref, s1_ref, u2_ref, dis_ref, b2_ref, o_ref):
  s2 = s0_ref[...] + s1_ref[...] + u2_ref[...]
  o_ref[...] = dis_ref[...] * s2 + b2_ref[...]


_layer2_out = pl.pallas_call(
    _layer2_out_body,
    grid=(N // ROWS_TC,),
    in_specs=[
        pl.BlockSpec((ROWS_TC, 1), lambda i: (i, 0)),
        pl.BlockSpec((ROWS_TC, 1), lambda i: (i, 0)),
        pl.BlockSpec((ROWS_TC, 1), lambda i: (i, 0)),
        pl.BlockSpec((ROWS_TC, 1), lambda i: (i, 0)),
        pl.BlockSpec((1, 1), lambda i: (0, 0)),
    ],
    out_specs=pl.BlockSpec((ROWS_TC, 1), lambda i: (i, 0)),
    out_shape=jax.ShapeDtypeStruct((N, 1), jnp.float32),
)

_deg_pass = _edge_pass(1, do_gather=False)
_agg16_pass = _edge_pass(D_HID, do_gather=True)
_agg1_pass = _edge_pass(1, do_gather=True)


@jax.jit
def kernel(x, edge_index, W1, b1, W2, b2):
  src = edge_index[0]
  dst = edge_index[1]
  pad = jnp.full((E_PAD - E,), N, dtype=jnp.int32)
  srcp = jnp.concatenate([src, pad])
  dstp = jnp.concatenate([dst, pad])

  ones1 = jnp.ones((BLK, 1), jnp.float32)
  zeros1 = jnp.zeros((RPT, 1), jnp.float32)
  zeros16 = jnp.zeros((RPT, D_HID), jnp.float32)

  degp = _deg_pass(dstp, ones1, zeros1)
  h = _matmul(x, W1)
  dis, u1 = _norm(degp[0, :N], degp[1, :N], h)

  s1p = _agg16_pass(u1, srcp, dstp, zeros16)

  u2 = _layer1_out(s1p[0], s1p[1], u1, dis,
                   b1.reshape(1, D_HID), W2.reshape(1, D_HID))

  s2p = _agg1_pass(u2, srcp, dstp, zeros1)

  return _layer2_out(s2p[0], s2p[1], u2, dis, b2.reshape(1, 1))


# speedup vs baseline: 27.9999x; 1.0055x over previous
import functools

import jax
import jax.numpy as jnp
from jax import lax
from jax.experimental import pallas as pl
from jax.experimental.pallas import tpu as pltpu
from jax.experimental.pallas import tpu_sc as plsc

N = 10000
E = 320000
D_IN = 128
D_HID = 16

NC = 2
NS = 16
NW = NC * NS
BLK = 128
EPW = 10240
E_PAD = EPW * NW
NB = EPW // BLK
N_ACC = 10240
RPT = N_ACC // NS
ROWS_TC = 1000


def _edge_pass(width: int, do_gather: bool):
  K = 8
  BLK_G = K * BLK
  NG = EPW // BLK_G

  def body(*refs):
    if do_gather:
      (table_hbm, src_hbm, dst_hbm, zeros_hbm, out_hbm, *rest) = refs
      banks = [(rest[0:K], rest[2 * K:3 * K], rest[4 * K:5 * K]),
               (rest[K:2 * K], rest[3 * K:4 * K], rest[5 * K:6 * K])]
      zbuf, acc, gsem, ssemA, ssemB = rest[6 * K:]
    else:
      (dst_hbm, ones_hbm, zeros_hbm, out_hbm, *rest) = refs
      banks = [(None, rest[0:K], None), (None, rest[K:2 * K], None)]
      ones_v = rest[2 * K]
      zbuf, acc, ssemA, ssemB = rest[2 * K + 1:]
    ssems = [ssemA, ssemB]
    c = lax.axis_index("c")
    s = lax.axis_index("s")
    wid = s * NC + c

    pltpu.sync_copy(zeros_hbm, zbuf)
    pltpu.sync_copy(zbuf, acc.at[pl.ds(s * RPT, RPT)])
    if not do_gather:
      pltpu.sync_copy(ones_hbm, ones_v)
    plsc.subcore_barrier()

    def prep(bank, g):
      sb, db, rw = bank
      base = wid * EPW + g * BLK_G
      if do_gather:
        for b in range(K):
          pltpu.sync_copy(src_hbm.at[pl.ds(base + b * BLK, BLK)], sb[b])
        for b in range(K):
          pltpu.async_copy(table_hbm.at[sb[b]], rw[b], gsem)
      for b in range(K):
        pltpu.sync_copy(dst_hbm.at[pl.ds(base + b * BLK, BLK)], db[b])
      if do_gather:
        for b in range(K):
          pltpu.make_async_copy(table_hbm.at[sb[b]], rw[b], gsem).wait()

    def fire(i):
      _, db, rw = banks[i]
      for b in range(K):
        pltpu.async_copy(rw[b] if do_gather else ones_v,
                         acc.at[db[b]], ssems[i], add=True)

    def drain(i):
      _, db, rw = banks[i]
      for b in range(K):
        pltpu.make_async_copy(rw[b] if do_gather else ones_v,
                              acc.at[db[b]], ssems[i]).wait()

    @pl.loop(0, NG // 2)
    def _(i):
      prep(banks[0], 2 * i)

      @pl.when(i > 0)
      def _():
        drain(1)

      fire(0)
      prep(banks[1], 2 * i + 1)
      drain(0)
      fire(1)

    drain(1)

    plsc.subcore_barrier()
    pltpu.sync_copy(acc.at[pl.ds(s * RPT, RPT)],
                    out_hbm.at[c, pl.ds(s * RPT, RPT)])

  scratch = []
  if do_gather:
    scratch += [pltpu.VMEM((BLK,), jnp.int32) for _ in range(2 * K)]
  scratch += [pltpu.VMEM((BLK,), jnp.int32) for _ in range(2 * K)]
  if do_gather:
    scratch += [pltpu.VMEM((BLK, width), jnp.float32)
                for _ in range(2 * K)]
  else:
    scratch += [pltpu.VMEM((BLK, width), jnp.float32)]
  scratch += [
      pltpu.VMEM((RPT, width), jnp.float32),
      pltpu.VMEM_SHARED((N_ACC, width), jnp.float32),
  ]
  if do_gather:
    scratch.append(pltpu.SemaphoreType.DMA)
  scratch += [pltpu.SemaphoreType.DMA, pltpu.SemaphoreType.DMA]
  return pl.kernel(
      body,
      out_type=jax.ShapeDtypeStruct((NC, N_ACC, width), jnp.float32),
      mesh=plsc.VectorSubcoreMesh(core_axis_name="c", subcore_axis_name="s"),
      scratch_types=scratch,
      compiler_params=pltpu.CompilerParams(use_tc_tiling_on_sc=False),
  )


def _mm_norm_body(x_ref, w_ref, d0_ref, d1_ref, dis_ref, u1_ref):
  h = jnp.dot(x_ref[...], w_ref[...], preferred_element_type=jnp.float32)
  deg = d0_ref[...] + d1_ref[...] + 1.0
  dis = lax.rsqrt(deg)
  dis_ref[...] = dis
  u1_ref[...] = h * dis


_mm_norm = pl.pallas_call(
    _mm_norm_body,
    grid=(N // ROWS_TC,),
    in_specs=[
        pl.BlockSpec((ROWS_TC, D_IN), lambda i: (i, 0)),
        pl.BlockSpec((D_IN, D_HID), lambda i: (0, 0)),
        pl.BlockSpec((ROWS_TC, 1), lambda i: (i, 0)),
        pl.BlockSpec((ROWS_TC, 1), lambda i: (i, 0)),
    ],
    out_specs=[
        pl.BlockSpec((ROWS_TC, 1), lambda i: (i, 0)),
        pl.BlockSpec((ROWS_TC, D_HID), lambda i: (i, 0)),
    ],
    out_shape=[
        jax.ShapeDtypeStruct((N, 1), jnp.float32),
        jax.ShapeDtypeStruct((N_ACC, D_HID), jnp.float32),
    ],
)


def _layer1_out_body(s0_ref, s1_ref, u1_ref, dis_ref, b1_ref, w2_ref, u2_ref):
  s1 = s0_ref[...] + s1_ref[...] + u1_ref[...]
  out1 = dis_ref[...] * s1 + b1_ref[...]
  h1 = jnp.maximum(out1, 0.0)
  z = jnp.sum(h1 * w2_ref[...], axis=1, keepdims=True)
  u2_ref[...] = dis_ref[...] * z


_layer1_out = pl.pallas_call(
    _layer1_out_body,
    grid=(N // ROWS_TC,),
    in_specs=[
        pl.BlockSpec((ROWS_TC, D_HID), lambda i: (i, 0)),
        pl.BlockSpec((ROWS_TC, D_HID), lambda i: (i, 0)),
        pl.BlockSpec((ROWS_TC, D_HID), lambda i: (i, 0)),
        pl.BlockSpec((ROWS_TC, 1), lambda i: (i, 0)),
        pl.BlockSpec((1, D_HID), lambda i: (0, 0)),
        pl.BlockSpec((1, D_HID), lambda i: (0, 0)),
    ],
    out_specs=pl.BlockSpec((ROWS_TC, 1), lambda i: (i, 0)),
    out_shape=jax.ShapeDtypeStruct((N_ACC, 1), jnp.float32),
)


def _layer2_out_body(s0_ref, s1_ref, u2_ref, dis_ref, b2_ref, o_ref):
  s2 = s0_ref[...] + s1_ref[...] + u2_ref[...]
  o_ref[...] = dis_ref[...] * s2 + b2_ref[...]


_layer2_out = pl.pallas_call(
    _layer2_out_body,
    grid=(N // ROWS_TC,),
    in_specs=[
        pl.BlockSpec((ROWS_TC, 1), lambda i: (i, 0)),
        pl.BlockSpec((ROWS_TC, 1), lambda i: (i, 0)),
        pl.BlockSpec((ROWS_TC, 1), lambda i: (i, 0)),
        pl.BlockSpec((ROWS_TC, 1), lambda i: (i, 0)),
        pl.BlockSpec((1, 1), lambda i: (0, 0)),
    ],
    out_specs=pl.BlockSpec((ROWS_TC, 1), lambda i: (i, 0)),
    out_shape=jax.ShapeDtypeStruct((N, 1), jnp.float32),
)

_deg_pass = _edge_pass(1, do_gather=False)
_agg16_pass = _edge_pass(D_HID, do_gather=True)
_agg1_pass = _edge_pass(1, do_gather=True)


@jax.jit
def kernel(x, edge_index, W1, b1, W2, b2):
  src = edge_index[0]
  dst = edge_index[1]
  pad = jnp.full((E_PAD - E,), N, dtype=jnp.int32)
  srcp = jnp.concatenate([src, pad])
  dstp = jnp.concatenate([dst, pad])

  ones1 = jnp.ones((BLK, 1), jnp.float32)
  zeros1 = jnp.zeros((RPT, 1), jnp.float32)
  zeros16 = jnp.zeros((RPT, D_HID), jnp.float32)

  degp = _deg_pass(dstp, ones1, zeros1)
  dis, u1 = _mm_norm(x, W1, degp[0, :N], degp[1, :N])

  s1p = _agg16_pass(u1, srcp, dstp, zeros16)

  u2 = _layer1_out(s1p[0], s1p[1], u1, dis,
                   b1.reshape(1, D_HID), W2.reshape(1, D_HID))

  s2p = _agg1_pass(u2, srcp, dstp, zeros1)

  return _layer2_out(s2p[0], s2p[1], u2, dis, b2.reshape(1, 1))
